# baseline (speedup 1.0000x reference)
import functools

import jax
import jax.numpy as jnp
from jax import lax
from jax.experimental import pallas as pl
from jax.experimental.pallas import tpu as pltpu
from jax.experimental.pallas import tpu_sc as plsc

N_NODES = 10000
E_EDGES = 320000
D_IN = 128
H_DIM = 64
C_OUT = 6
G_GRAPHS = 64

NUM_CORES = 2
NUM_SUBCORES = 16
NUM_WORKERS = NUM_CORES * NUM_SUBCORES

NP = 10240
ASTRIPE = NP // NUM_SUBCORES
TSTRIPE = N_NODES // NUM_SUBCORES
CHUNK = 80
SUB = 5
BLK = SUB * CHUNK
NBLK = 25
EPT = NBLK * BLK

_mesh = plsc.VectorSubcoreMesh(core_axis_name="c", subcore_axis_name="s")


@functools.partial(
    pl.kernel,
    mesh=_mesh,
    out_type=jax.ShapeDtypeStruct((NUM_CORES, NP), jnp.float32),
    scratch_types=[
        pltpu.VMEM((4, SUB, CHUNK), jnp.int32),
        pltpu.VMEM((CHUNK,), jnp.float32),
        pltpu.VMEM((ASTRIPE,), jnp.float32),
        pltpu.VMEM_SHARED((NP,), jnp.float32),
        pltpu.SemaphoreType.DMA,
        pltpu.SemaphoreType.DMA,
        pltpu.SemaphoreType.DMA,
        pltpu.SemaphoreType.DMA,
        pltpu.SemaphoreType.DMA,
        pltpu.SemaphoreType.DMA,
    ],
    compiler_params=pltpu.CompilerParams(use_tc_tiling_on_sc=False),
)
def _sc_degree(edge_hbm, out_hbm, dbuf, ones_v, zero_v, acc_sh,
               isem0, isem1, isem2, isem3, ssem0, ssem1):
    c = lax.axis_index("c")
    s = lax.axis_index("s")
    wid = s * NUM_CORES + c
    e0 = wid * EPT
    isem = (isem0, isem1, isem2, isem3)
    ssem = (ssem0, ssem1)

    one16 = jnp.full((16,), 1.0, jnp.float32)
    zero16 = jnp.zeros((16,), jnp.float32)
    for j in range(CHUNK // 16):
        ones_v[pl.ds(j * 16, 16)] = one16

    def zfill(i, carry):
        zero_v[pl.ds(i * 16, 16)] = zero16
        return carry

    lax.fori_loop(0, ASTRIPE // 16, zfill, 0)
    pltpu.sync_copy(zero_v, acc_sh.at[pl.ds(s * ASTRIPE, ASTRIPE)])
    plsc.subcore_barrier()

    def start_idx(b):
        return [pltpu.async_copy(
                    edge_hbm.at[1, pl.ds(e0 + b * BLK + j * CHUNK, CHUNK)],
                    dbuf.at[b % 4, j], isem[b % 4])
                for j in range(SUB)]

    h_idx = {b: start_idx(b) for b in range(min(3, NBLK))}
    pend = {}
    for b in range(NBLK):
        if b >= 1:
            for h in pend.pop(b - 1):
                h.wait()
        if b + 3 < NBLK:
            h_idx[b + 3] = start_idx(b + 3)
        for h in h_idx.pop(b):
            h.wait()
        pend[b] = [pltpu.async_copy(ones_v, acc_sh.at[dbuf.at[b % 4, j]],
                                    ssem[b % 2], add=True)
                   for j in range(SUB)]
    for h in pend.pop(NBLK - 1):
        h.wait()
    plsc.subcore_barrier()
    pltpu.sync_copy(acc_sh.at[pl.ds(s * ASTRIPE, ASTRIPE)],
                    out_hbm.at[c, pl.ds(s * ASTRIPE, ASTRIPE)])


@functools.partial(
    pl.kernel,
    mesh=_mesh,
    out_type=jax.ShapeDtypeStruct((NUM_CORES, NP, 128), jnp.bfloat16),
    scratch_types=[
        pltpu.VMEM((4, 2 * SUB, CHUNK), jnp.int32),
        pltpu.VMEM((2, SUB, CHUNK, H_DIM), jnp.bfloat16),
        pltpu.VMEM((ASTRIPE // 5, H_DIM), jnp.bfloat16),
        pltpu.VMEM_SHARED((NP, H_DIM), jnp.bfloat16),
        pltpu.VMEM_SHARED((NP, H_DIM), jnp.bfloat16),
        pltpu.SemaphoreType.DMA,
        pltpu.SemaphoreType.DMA,
        pltpu.SemaphoreType.DMA,
        pltpu.SemaphoreType.DMA,
        pltpu.SemaphoreType.DMA,
        pltpu.SemaphoreType.DMA,
        pltpu.SemaphoreType.DMA,
        pltpu.SemaphoreType.DMA,
    ],
    compiler_params=pltpu.CompilerParams(use_tc_tiling_on_sc=False),
)
def _sc_edge_pass(u_hbm, edge_hbm, out_hbm, ibuf, rows, zrow, acc_sh, u_sh,
                  isem0, isem1, isem2, isem3, gsem0, gsem1, ssem0, ssem1):
    c = lax.axis_index("c")
    s = lax.axis_index("s")
    wid = s * NUM_CORES + c
    e0 = wid * EPT
    isem = (isem0, isem1, isem2, isem3)
    gsem = (gsem0, gsem1)
    ssem = (ssem0, ssem1)

    zero32 = jnp.zeros((32,), jnp.bfloat16)

    def zfill(i, carry):
        for j in range(H_DIM // 32):
            zrow[i, pl.ds(j * 32, 32)] = zero32
        return carry

    lax.fori_loop(0, ASTRIPE // 5, zfill, 0)
    for t in range(5):
        pltpu.sync_copy(
            zrow, acc_sh.at[pl.ds(s * ASTRIPE + t * (ASTRIPE // 5),
                                  ASTRIPE // 5)])
    pltpu.sync_copy(u_hbm.at[pl.ds(s * ASTRIPE, ASTRIPE), pl.ds(0, H_DIM)],
                    u_sh.at[pl.ds(s * ASTRIPE, ASTRIPE)])
    plsc.subcore_barrier()

    def start_idx(b):
        base = e0 + b * BLK
        return [pltpu.async_copy(
                    edge_hbm.at[r, pl.ds(base + j * CHUNK, CHUNK)],
                    ibuf.at[b % 4, r * SUB + j], isem[b % 4])
                for r in range(2) for j in range(SUB)]

    def fire_gathers(b):
        return [pltpu.async_copy(u_sh.at[ibuf.at[b % 4, j]],
                                 rows.at[b % 2, j], gsem[b % 2])
                for j in range(SUB)]

    def fire_scatters(b):
        return [pltpu.async_copy(rows.at[b % 2, j],
                                 acc_sh.at[ibuf.at[b % 4, SUB + j]],
                                 ssem[b % 2], add=True)
                for j in range(SUB)]

    h_idx = {b: start_idx(b) for b in range(min(3, NBLK))}
    for h in h_idx.pop(0):
        h.wait()
    pend_g = {0: fire_gathers(0)}
    pend_s = {}
    for b in range(NBLK):
        if b >= 1:
            for h in pend_s.pop(b - 1):
                h.wait()
        if b + 3 < NBLK:
            h_idx[b + 3] = start_idx(b + 3)
        if b + 1 < NBLK:
            for h in h_idx.pop(b + 1):
                h.wait()
            pend_g[b + 1] = fire_gathers(b + 1)
        for h in pend_g.pop(b):
            h.wait()
        pend_s[b] = fire_scatters(b)
    for h in pend_s.pop(NBLK - 1):
        h.wait()
    plsc.subcore_barrier()
    pltpu.sync_copy(acc_sh.at[pl.ds(s * ASTRIPE, ASTRIPE)],
                    out_hbm.at[c, pl.ds(s * ASTRIPE, ASTRIPE),
                               pl.ds(0, H_DIM)])


def _tc_first_body(x_ref, w1_ref, deg_ref, u_ref, dinv_ref):
    deg = deg_ref[0][:N_NODES, None] + deg_ref[1][:N_NODES, None] + 1.0
    dinv = lax.rsqrt(deg)
    z = jnp.dot(x_ref[...], w1_ref[...], preferred_element_type=jnp.float32)
    u_ref[:N_NODES, :H_DIM] = (z * dinv).astype(jnp.bfloat16)
    dinv_ref[...] = dinv


def _tc_mid_body(p_ref, u1_ref, dinv_ref, b1_ref, w2_ref, u2_ref):
    ssum = (p_ref[0, :N_NODES, :H_DIM].astype(jnp.float32)
            + p_ref[1, :N_NODES, :H_DIM].astype(jnp.float32)
            + u1_ref[:N_NODES, :H_DIM].astype(jnp.float32))
    h = jnp.maximum(ssum * dinv_ref[...] + b1_ref[...], 0.0)
    u2_ref[:N_NODES, :H_DIM] = (jnp.dot(h * dinv_ref[...], w2_ref[...],
                                        preferred_element_type=jnp.float32)
                                ).astype(jnp.bfloat16)


def _tc_tail_body(p_ref, u2_ref, dinv_ref, b2_ref, batch_ref, w3_ref, b3_ref,
                  out_ref):
    ssum = (p_ref[0, :N_NODES, :H_DIM].astype(jnp.float32)
            + p_ref[1, :N_NODES, :H_DIM].astype(jnp.float32)
            + u2_ref[:N_NODES, :H_DIM].astype(jnp.float32))
    h = jnp.maximum(ssum * dinv_ref[...] + b2_ref[...], 0.0)
    gid = lax.broadcasted_iota(jnp.int32, (N_NODES, G_GRAPHS), 1)
    onehot = (batch_ref[...] == gid).astype(jnp.float32)
    gsum = lax.dot_general(onehot, h, (((0,), (0,)), ((), ())),
                           preferred_element_type=jnp.float32)
    cnt = jnp.sum(onehot, axis=0)[:, None]
    g = gsum / jnp.maximum(cnt, 1.0)
    out_ref[...] = jnp.dot(g, w3_ref[...],
                           preferred_element_type=jnp.float32) + b3_ref[...]


_tc_first = pl.pallas_call(
    _tc_first_body,
    out_shape=(jax.ShapeDtypeStruct((NP, 128), jnp.bfloat16),
               jax.ShapeDtypeStruct((N_NODES, 1), jnp.float32)),
)

_tc_mid = pl.pallas_call(
    _tc_mid_body,
    out_shape=jax.ShapeDtypeStruct((NP, 128), jnp.bfloat16),
)

_tc_tail = pl.pallas_call(
    _tc_tail_body,
    out_shape=jax.ShapeDtypeStruct((G_GRAPHS, C_OUT), jnp.float32),
)


def kernel(x, edge_index, batch, W1, b1, W2, b2, W3, b3):
    deg_parts = _sc_degree(edge_index)
    u1, dinv = _tc_first(x, W1, deg_parts)
    p1 = _sc_edge_pass(u1, edge_index)
    u2 = _tc_mid(p1, u1, dinv, b1[None, :], W2)
    p2 = _sc_edge_pass(u2, edge_index)
    return _tc_tail(p2, u2, dinv, b2[None, :], batch[:, None], W3,
                    b3[None, :])


# speedup vs baseline: 1.2229x; 1.2229x over previous
import functools

import jax
import jax.numpy as jnp
from jax import lax
from jax.experimental import pallas as pl
from jax.experimental.pallas import tpu as pltpu
from jax.experimental.pallas import tpu_sc as plsc

N_NODES = 10000
E_EDGES = 320000
D_IN = 128
H_DIM = 64
C_OUT = 6
G_GRAPHS = 64

NUM_CORES = 2
NUM_SUBCORES = 16
NUM_WORKERS = NUM_CORES * NUM_SUBCORES

NP = 10240
ASTRIPE = NP // NUM_SUBCORES
TSTRIPE = N_NODES // NUM_SUBCORES
CHUNK = 80
SUB = 5
BLK = SUB * CHUNK
NBLK = 25
EPT = NBLK * BLK

_mesh = plsc.VectorSubcoreMesh(core_axis_name="c", subcore_axis_name="s")


@functools.partial(
    pl.kernel,
    mesh=_mesh,
    out_type=jax.ShapeDtypeStruct((NUM_CORES, NP), jnp.float32),
    scratch_types=[
        pltpu.VMEM((4, SUB, CHUNK), jnp.int32),
        pltpu.VMEM((CHUNK,), jnp.float32),
        pltpu.VMEM((ASTRIPE,), jnp.float32),
        pltpu.VMEM_SHARED((NP,), jnp.float32),
        pltpu.SemaphoreType.DMA,
        pltpu.SemaphoreType.DMA,
        pltpu.SemaphoreType.DMA,
        pltpu.SemaphoreType.DMA,
        pltpu.SemaphoreType.DMA,
        pltpu.SemaphoreType.DMA,
    ],
    compiler_params=pltpu.CompilerParams(use_tc_tiling_on_sc=False),
)
def _sc_degree(edge_hbm, out_hbm, dbuf, ones_v, zero_v, acc_sh,
               isem0, isem1, isem2, isem3, ssem0, ssem1):
    c = lax.axis_index("c")
    s = lax.axis_index("s")
    wid = s * NUM_CORES + c
    e0 = wid * EPT
    isem = (isem0, isem1, isem2, isem3)
    ssem = (ssem0, ssem1)

    one16 = jnp.full((16,), 1.0, jnp.float32)
    zero16 = jnp.zeros((16,), jnp.float32)
    for j in range(CHUNK // 16):
        ones_v[pl.ds(j * 16, 16)] = one16

    def zfill(i, carry):
        zero_v[pl.ds(i * 16, 16)] = zero16
        return carry

    lax.fori_loop(0, ASTRIPE // 16, zfill, 0)
    pltpu.sync_copy(zero_v, acc_sh.at[pl.ds(s * ASTRIPE, ASTRIPE)])
    plsc.subcore_barrier()

    def start_idx(b):
        return [pltpu.async_copy(
                    edge_hbm.at[1, pl.ds(e0 + b * BLK + j * CHUNK, CHUNK)],
                    dbuf.at[b % 4, j], isem[b % 4])
                for j in range(SUB)]

    h_idx = {b: start_idx(b) for b in range(min(3, NBLK))}
    pend = {}
    for b in range(NBLK):
        if b >= 1:
            for h in pend.pop(b - 1):
                h.wait()
        if b + 3 < NBLK:
            h_idx[b + 3] = start_idx(b + 3)
        for h in h_idx.pop(b):
            h.wait()
        pend[b] = [pltpu.async_copy(ones_v, acc_sh.at[dbuf.at[b % 4, j]],
                                    ssem[b % 2], add=True)
                   for j in range(SUB)]
    for h in pend.pop(NBLK - 1):
        h.wait()
    plsc.subcore_barrier()
    pltpu.sync_copy(acc_sh.at[pl.ds(s * ASTRIPE, ASTRIPE)],
                    out_hbm.at[c, pl.ds(s * ASTRIPE, ASTRIPE)])


@functools.partial(
    pl.kernel,
    mesh=_mesh,
    out_type=jax.ShapeDtypeStruct((NUM_CORES, NP, H_DIM), jnp.bfloat16),
    scratch_types=[
        pltpu.VMEM((4, 2 * SUB, CHUNK), jnp.int32),
        pltpu.VMEM((2, SUB, CHUNK, H_DIM), jnp.bfloat16),
        pltpu.VMEM((ASTRIPE // 5, H_DIM), jnp.bfloat16),
        pltpu.VMEM_SHARED((NP, H_DIM), jnp.bfloat16),
        pltpu.VMEM_SHARED((N_NODES, H_DIM), jnp.bfloat16),
        pltpu.SemaphoreType.DMA,
        pltpu.SemaphoreType.DMA,
        pltpu.SemaphoreType.DMA,
        pltpu.SemaphoreType.DMA,
        pltpu.SemaphoreType.DMA,
        pltpu.SemaphoreType.DMA,
        pltpu.SemaphoreType.DMA,
        pltpu.SemaphoreType.DMA,
    ],
    compiler_params=pltpu.CompilerParams(use_tc_tiling_on_sc=False),
)
def _sc_edge_pass(u_hbm, edge_hbm, out_hbm, ibuf, rows, zrow, acc_sh, u_sh,
                  isem0, isem1, isem2, isem3, gsem0, gsem1, ssem0, ssem1):
    c = lax.axis_index("c")
    s = lax.axis_index("s")
    wid = s * NUM_CORES + c
    e0 = wid * EPT
    isem = (isem0, isem1, isem2, isem3)
    gsem = (gsem0, gsem1)
    ssem = (ssem0, ssem1)

    zero32 = jnp.zeros((32,), jnp.bfloat16)

    def zfill(i, carry):
        for j in range(H_DIM // 32):
            zrow[i, pl.ds(j * 32, 32)] = zero32
        return carry

    lax.fori_loop(0, ASTRIPE // 5, zfill, 0)
    for t in range(5):
        pltpu.sync_copy(
            zrow, acc_sh.at[pl.ds(s * ASTRIPE + t * (ASTRIPE // 5),
                                  ASTRIPE // 5)])
    pltpu.sync_copy(u_hbm.at[pl.ds(s * TSTRIPE, TSTRIPE)],
                    u_sh.at[pl.ds(s * TSTRIPE, TSTRIPE)])
    plsc.subcore_barrier()

    def start_idx(b):
        base = e0 + b * BLK
        return [pltpu.async_copy(
                    edge_hbm.at[r, pl.ds(base + j * CHUNK, CHUNK)],
                    ibuf.at[b % 4, r * SUB + j], isem[b % 4])
                for r in range(2) for j in range(SUB)]

    def fire_gathers(b):
        return [pltpu.async_copy(u_sh.at[ibuf.at[b % 4, j]],
                                 rows.at[b % 2, j], gsem[b % 2])
                for j in range(SUB)]

    def fire_scatters(b):
        return [pltpu.async_copy(rows.at[b % 2, j],
                                 acc_sh.at[ibuf.at[b % 4, SUB + j]],
                                 ssem[b % 2], add=True)
                for j in range(SUB)]

    h_idx = {b: start_idx(b) for b in range(min(3, NBLK))}
    for h in h_idx.pop(0):
        h.wait()
    pend_g = {0: fire_gathers(0)}
    pend_s = {}
    for b in range(NBLK):
        if b >= 1:
            for h in pend_s.pop(b - 1):
                h.wait()
        if b + 3 < NBLK:
            h_idx[b + 3] = start_idx(b + 3)
        if b + 1 < NBLK:
            for h in h_idx.pop(b + 1):
                h.wait()
            pend_g[b + 1] = fire_gathers(b + 1)
        for h in pend_g.pop(b):
            h.wait()
        pend_s[b] = fire_scatters(b)
    for h in pend_s.pop(NBLK - 1):
        h.wait()
    plsc.subcore_barrier()
    pltpu.sync_copy(acc_sh.at[pl.ds(s * ASTRIPE, ASTRIPE)],
                    out_hbm.at[c, pl.ds(s * ASTRIPE, ASTRIPE)])


def _tc_first_body(x_ref, w1_ref, deg_ref, u_ref, dinv_ref):
    deg = deg_ref[0][:N_NODES, None] + deg_ref[1][:N_NODES, None] + 1.0
    dinv = lax.rsqrt(deg)
    z = jnp.dot(x_ref[...], w1_ref[...], preferred_element_type=jnp.float32)
    u_ref[...] = (z * dinv).astype(jnp.bfloat16)
    dinv_ref[...] = dinv


def _tc_mid_body(p_ref, u1_ref, dinv_ref, b1_ref, w2_ref, u2_ref):
    ssum = (p_ref[0, :N_NODES].astype(jnp.float32)
            + p_ref[1, :N_NODES].astype(jnp.float32)
            + u1_ref[...].astype(jnp.float32))
    h = jnp.maximum(ssum * dinv_ref[...] + b1_ref[...], 0.0)
    u2_ref[...] = (jnp.dot(h * dinv_ref[...], w2_ref[...],
                           preferred_element_type=jnp.float32)
                   ).astype(jnp.bfloat16)


def _tc_tail_body(p_ref, u2_ref, dinv_ref, b2_ref, batch_ref, w3_ref, b3_ref,
                  out_ref):
    ssum = (p_ref[0, :N_NODES].astype(jnp.float32)
            + p_ref[1, :N_NODES].astype(jnp.float32)
            + u2_ref[...].astype(jnp.float32))
    h = jnp.maximum(ssum * dinv_ref[...] + b2_ref[...], 0.0)
    gid = lax.broadcasted_iota(jnp.int32, (N_NODES, G_GRAPHS), 1)
    onehot = (batch_ref[...] == gid).astype(jnp.float32)
    gsum = lax.dot_general(onehot, h, (((0,), (0,)), ((), ())),
                           preferred_element_type=jnp.float32)
    cnt = jnp.sum(onehot, axis=0)[:, None]
    g = gsum / jnp.maximum(cnt, 1.0)
    out_ref[...] = jnp.dot(g, w3_ref[...],
                           preferred_element_type=jnp.float32) + b3_ref[...]


_tc_first = pl.pallas_call(
    _tc_first_body,
    out_shape=(jax.ShapeDtypeStruct((N_NODES, H_DIM), jnp.bfloat16),
               jax.ShapeDtypeStruct((N_NODES, 1), jnp.float32)),
)

_tc_mid = pl.pallas_call(
    _tc_mid_body,
    out_shape=jax.ShapeDtypeStruct((N_NODES, H_DIM), jnp.bfloat16),
)

_tc_tail = pl.pallas_call(
    _tc_tail_body,
    out_shape=jax.ShapeDtypeStruct((G_GRAPHS, C_OUT), jnp.float32),
)


def kernel(x, edge_index, batch, W1, b1, W2, b2, W3, b3):
    deg_parts = _sc_degree(edge_index)
    u1, dinv = _tc_first(x, W1, deg_parts)
    p1 = _sc_edge_pass(u1, edge_index)
    u2 = _tc_mid(p1, u1, dinv, b1[None, :], W2)
    p2 = _sc_edge_pass(u2, edge_index)
    return _tc_tail(p2, u2, dinv, b2[None, :], batch[:, None], W3,
                    b3[None, :])
